# Initial kernel scaffold; baseline (speedup 1.0000x reference)
import jax
import jax.numpy as jnp
from jax.experimental import pallas as pl


def kernel(x, edge_index, batch, W1, b1, W2, b2, Wc, bc):
    raise NotImplementedError("write your pallas kernel here")


# speedup vs baseline: 13.2827x; 13.2827x over previous
import functools

import jax
import jax.numpy as jnp
from jax import lax
from jax.experimental import pallas as pl
from jax.experimental.pallas import tpu as pltpu
from jax.experimental.pallas import tpu_sc as plsc

N = 10000
E = 320000
D = 128
H = 128
B = 64

NC = 2
NS = 16
NW = NC * NS
NPAD = 10240
RPS = NPAD // NS
EPW = E // NW
K = 80

_f32 = jnp.float32


def _sc_degree(dst, zeros_n):
    mesh = plsc.VectorSubcoreMesh(core_axis_name="c", subcore_axis_name="s")

    @functools.partial(
        pl.kernel,
        out_type=(jax.ShapeDtypeStruct((NPAD,), _f32),
                  jax.ShapeDtypeStruct((NPAD,), _f32)),
        mesh=mesh,
        scratch_types=[
            pltpu.VMEM_SHARED((NPAD,), _f32),
            pltpu.VMEM((K,), jnp.int32),
            pltpu.VMEM((K,), _f32),
        ],
    )
    def k(dst_hbm, z_hbm, out_a, out_b, acc, idx_v, ones_v):
        cid = lax.axis_index("c")
        sid = lax.axis_index("s")
        wid = sid * NC + cid
        for i in range(K // 16):
            ones_v[pl.ds(i * 16, 16)] = jnp.full((16,), 1.0, _f32)
        pltpu.sync_copy(z_hbm.at[pl.ds(sid * RPS, RPS)],
                        acc.at[pl.ds(sid * RPS, RPS)])
        plsc.subcore_barrier()

        base = wid * EPW

        def body(t, carry):
            pltpu.sync_copy(dst_hbm.at[pl.ds(base + t * K, K)], idx_v)
            pltpu.sync_copy(ones_v, acc.at[idx_v], add=True)
            return carry

        lax.fori_loop(0, EPW // K, body, 0)
        plsc.subcore_barrier()

        @pl.when(cid == 0)
        def _():
            pltpu.sync_copy(acc.at[pl.ds(sid * RPS, RPS)],
                            out_a.at[pl.ds(sid * RPS, RPS)])

        @pl.when(cid == 1)
        def _():
            pltpu.sync_copy(acc.at[pl.ds(sid * RPS, RPS)],
                            out_b.at[pl.ds(sid * RPS, RPS)])

    return k(dst, zeros_n)


def _sc_scatter_rows(hs, src, dst, zeros_rows):
    mesh = plsc.VectorSubcoreMesh(core_axis_name="c", subcore_axis_name="s")

    @functools.partial(
        pl.kernel,
        out_type=(jax.ShapeDtypeStruct((NPAD, H), _f32),
                  jax.ShapeDtypeStruct((NPAD, H), _f32)),
        mesh=mesh,
        scratch_types=[
            pltpu.VMEM_SHARED((NPAD, H), _f32),
            pltpu.VMEM((K,), jnp.int32),
            pltpu.VMEM((K,), jnp.int32),
            pltpu.VMEM((K, H), _f32),
            pltpu.SemaphoreType.DMA,
        ],
    )
    def k(hs_hbm, src_hbm, dst_hbm, z_hbm, out_a, out_b,
          acc, src_v, dst_v, rows_v, sem):
        cid = lax.axis_index("c")
        sid = lax.axis_index("s")
        wid = sid * NC + cid
        pltpu.sync_copy(z_hbm.at[pl.ds(sid * RPS, RPS)],
                        acc.at[pl.ds(sid * RPS, RPS)])
        plsc.subcore_barrier()

        base = wid * EPW

        def body(t, carry):
            off = base + t * K
            pltpu.sync_copy(src_hbm.at[pl.ds(off, K)], src_v)
            pltpu.sync_copy(dst_hbm.at[pl.ds(off, K)], dst_v)
            pltpu.async_copy(hs_hbm.at[src_v], rows_v, sem).wait()
            pltpu.sync_copy(rows_v, acc.at[dst_v], add=True)
            return carry

        lax.fori_loop(0, EPW // K, body, 0)
        plsc.subcore_barrier()

        @pl.when(cid == 0)
        def _():
            pltpu.sync_copy(acc.at[pl.ds(sid * RPS, RPS)],
                            out_a.at[pl.ds(sid * RPS, RPS)])

        @pl.when(cid == 1)
        def _():
            pltpu.sync_copy(acc.at[pl.ds(sid * RPS, RPS)],
                            out_b.at[pl.ds(sid * RPS, RPS)])

    return k(hs, src, dst, zeros_rows)


_R = 2048
_GRID = NPAD // _R


def _tc1(degt, xp, W1):
    def body(deg_ref, x_ref, w_ref, hs_ref, dinv_ref):
        d = deg_ref[:, 0:1] + deg_ref[:, 1:2] + 1.0
        di = lax.rsqrt(d)
        hw = jnp.dot(x_ref[...], w_ref[...], preferred_element_type=_f32)
        hs_ref[...] = hw * di
        dinv_ref[...] = di

    return pl.pallas_call(
        body,
        grid=(_GRID,),
        in_specs=[
            pl.BlockSpec((_R, 2), lambda i: (i, 0)),
            pl.BlockSpec((_R, D), lambda i: (i, 0)),
            pl.BlockSpec((D, H), lambda i: (0, 0)),
        ],
        out_specs=[
            pl.BlockSpec((_R, H), lambda i: (i, 0)),
            pl.BlockSpec((_R, 1), lambda i: (i, 0)),
        ],
        out_shape=[
            jax.ShapeDtypeStruct((NPAD, H), _f32),
            jax.ShapeDtypeStruct((NPAD, 1), _f32),
        ],
    )(degt, xp, W1)


def _tc2(s1a, s1b, hs1, dinv, b1r, W2):
    def body(sa_ref, sb_ref, hs_ref, di_ref, b_ref, w_ref, out_ref):
        di = di_ref[...]
        h1 = di * (sa_ref[...] + sb_ref[...] + hs_ref[...]) + b_ref[...]
        h1 = jnp.maximum(h1, 0.0)
        out_ref[...] = di * jnp.dot(h1, w_ref[...],
                                    preferred_element_type=_f32)

    return pl.pallas_call(
        body,
        grid=(_GRID,),
        in_specs=[
            pl.BlockSpec((_R, H), lambda i: (i, 0)),
            pl.BlockSpec((_R, H), lambda i: (i, 0)),
            pl.BlockSpec((_R, H), lambda i: (i, 0)),
            pl.BlockSpec((_R, 1), lambda i: (i, 0)),
            pl.BlockSpec((1, H), lambda i: (0, 0)),
            pl.BlockSpec((H, H), lambda i: (0, 0)),
        ],
        out_specs=pl.BlockSpec((_R, H), lambda i: (i, 0)),
        out_shape=jax.ShapeDtypeStruct((NPAD, H), _f32),
    )(s1a, s1b, hs1, dinv, b1r, W2)


def _tc3(s2a, s2b, hs2, dinv, b2r, batp, Wc, bcr):
    def body(sa_ref, sb_ref, hs_ref, di_ref, b_ref, bat_ref, wc_ref, bc_ref,
             out_ref, psum, pcnt):
        i = pl.program_id(0)

        @pl.when(i == 0)
        def _():
            psum[...] = jnp.zeros_like(psum)
            pcnt[...] = jnp.zeros_like(pcnt)

        h2 = di_ref[...] * (sa_ref[...] + sb_ref[...] + hs_ref[...]) \
            + b_ref[...]
        m = (bat_ref[...] ==
             lax.broadcasted_iota(jnp.int32, (_R, B), 1)).astype(_f32)
        psum[...] += lax.dot_general(m, h2, (((0,), (0,)), ((), ())),
                                     preferred_element_type=_f32)
        pcnt[...] += lax.dot_general(m, jnp.ones((_R, 1), _f32),
                                     (((0,), (0,)), ((), ())),
                                     preferred_element_type=_f32)

        @pl.when(i == pl.num_programs(0) - 1)
        def _():
            pooled = psum[...] / jnp.maximum(pcnt[...], 1.0)
            out_ref[...] = jnp.dot(pooled, wc_ref[...],
                                   preferred_element_type=_f32) + bc_ref[...]

    return pl.pallas_call(
        body,
        grid=(_GRID,),
        in_specs=[
            pl.BlockSpec((_R, H), lambda i: (i, 0)),
            pl.BlockSpec((_R, H), lambda i: (i, 0)),
            pl.BlockSpec((_R, H), lambda i: (i, 0)),
            pl.BlockSpec((_R, 1), lambda i: (i, 0)),
            pl.BlockSpec((1, H), lambda i: (0, 0)),
            pl.BlockSpec((_R, 1), lambda i: (i, 0)),
            pl.BlockSpec((H, 1), lambda i: (0, 0)),
            pl.BlockSpec((1, 1), lambda i: (0, 0)),
        ],
        out_specs=pl.BlockSpec((B, 1), lambda i: (0, 0)),
        out_shape=jax.ShapeDtypeStruct((B, 1), _f32),
        scratch_shapes=[
            pltpu.VMEM((B, H), _f32),
            pltpu.VMEM((B, 1), _f32),
        ],
    )(s2a, s2b, hs2, dinv, b2r, batp, Wc, bcr)


def kernel(x, edge_index, batch, W1, b1, W2, b2, Wc, bc):
    src = edge_index[0]
    dst = edge_index[1]

    xp = jnp.zeros((NPAD, D), _f32).at[:N].set(x)
    batp = jnp.full((NPAD, 1), B + 1, jnp.int32).at[:N, 0].set(batch)
    zeros_rows = jnp.zeros((NPAD, H), _f32)
    zeros_n = jnp.zeros((NPAD,), _f32)

    deg_a, deg_b = _sc_degree(dst, zeros_n)
    degt = jnp.stack([deg_a, deg_b], axis=1)

    hs1, dinv = _tc1(degt, xp, W1)
    s1a, s1b = _sc_scatter_rows(hs1, src, dst, zeros_rows)
    hs2 = _tc2(s1a, s1b, hs1, dinv, b1.reshape(1, H), W2)
    s2a, s2b = _sc_scatter_rows(hs2, src, dst, zeros_rows)
    out = _tc3(s2a, s2b, hs2, dinv, b2.reshape(1, H), batp,
               Wc, bc.reshape(1, 1))
    return out
